# Initial kernel scaffold; baseline (speedup 1.0000x reference)
import jax
import jax.numpy as jnp
from jax.experimental import pallas as pl


def kernel(x, token_table, pos_table):
    raise NotImplementedError("write your pallas kernel here")


# speedup vs baseline: 2.4548x; 2.4548x over previous
import functools

import jax
import jax.numpy as jnp
import numpy as np
from jax import lax
from jax.experimental import pallas as pl
from jax.experimental.pallas import tpu as pltpu
from jax.experimental.pallas import tpu_sc as plsc

BATCH = 4096
SEQ = 200
EMBED = 64

_NC, _NS = 2, 16
_NW = _NC * _NS
_PER_W = (BATCH * SEQ) // _NW
_CHUNK = 128
_NCHUNK = _PER_W // _CHUNK

_POS_IDX = jnp.asarray(np.tile(np.arange(1, SEQ + 1, dtype=np.int32), _PER_W // SEQ))


def _sc_embed(x, token_table, pos_table):
    mesh = plsc.VectorSubcoreMesh(
        core_axis_name="c", subcore_axis_name="s", num_cores=_NC, num_subcores=_NS
    )

    @functools.partial(
        pl.kernel,
        mesh=mesh,
        out_type=jax.ShapeDtypeStruct((BATCH * SEQ, EMBED), jnp.float32),
        scratch_types=[
            pltpu.VMEM((_PER_W,), jnp.int32),
            pltpu.VMEM((_PER_W,), jnp.int32),
            pltpu.VMEM((_CHUNK, EMBED), jnp.float32),
            pltpu.SemaphoreType.DMA,
        ],
        compiler_params=pltpu.CompilerParams(use_tc_tiling_on_sc=False),
    )
    def k(x_hbm, tok_hbm, pos_hbm, pidx_hbm, out_hbm, idx_v, pidx_v, buf, sem):
        wid = lax.axis_index("s") * _NC + lax.axis_index("c")
        base = wid * _PER_W
        pltpu.sync_copy(x_hbm.at[pl.ds(base, _PER_W)], idx_v)
        pltpu.sync_copy(pidx_hbm, pidx_v)

        def body(c, carry):
            off = c * _CHUNK
            pltpu.async_copy(
                tok_hbm.at[idx_v.at[pl.ds(off, _CHUNK)]], buf, sem
            ).wait()
            pltpu.async_copy(
                pos_hbm.at[pidx_v.at[pl.ds(off, _CHUNK)]], buf, sem, add=True
            ).wait()
            pltpu.sync_copy(buf, out_hbm.at[pl.ds(base + off, _CHUNK)])
            return carry

        lax.fori_loop(0, _NCHUNK, body, 0)

    return k(x.reshape(BATCH * SEQ), token_table, pos_table, _POS_IDX)


def _mask_body(x_ref, m_ref):
    m_ref[...] = x_ref[...] != 0


def _mask(x):
    return pl.pallas_call(
        _mask_body,
        out_shape=jax.ShapeDtypeStruct((BATCH, SEQ), jnp.bool_),
        grid=(8,),
        in_specs=[pl.BlockSpec((BATCH // 8, SEQ), lambda i: (i, 0))],
        out_specs=pl.BlockSpec((BATCH // 8, SEQ), lambda i: (i, 0)),
    )(x)


def kernel(x, token_table, pos_table):
    out_flat = _sc_embed(x, token_table, pos_table)
    mask = _mask(x)
    return out_flat.reshape(BATCH, SEQ, EMBED), mask


# speedup vs baseline: 2.4898x; 1.0142x over previous
import functools

import jax
import jax.numpy as jnp
import numpy as np
from jax import lax
from jax.experimental import pallas as pl
from jax.experimental.pallas import tpu as pltpu
from jax.experimental.pallas import tpu_sc as plsc

BATCH = 4096
SEQ = 200
EMBED = 64

_NC, _NS = 2, 16
_NW = _NC * _NS
_PER_W = (BATCH * SEQ) // _NW
_TR = 128
_GROUP = 512
_NTR = _GROUP // _TR
_NG = _PER_W // _GROUP
_NSLOT = 2

_POS_IDX = jnp.asarray(np.tile(np.arange(1, SEQ + 1, dtype=np.int32), _PER_W // SEQ))


def _sc_embed(x, token_table, pos_table):
    mesh = plsc.VectorSubcoreMesh(
        core_axis_name="c", subcore_axis_name="s", num_cores=_NC, num_subcores=_NS
    )

    @functools.partial(
        pl.kernel,
        mesh=mesh,
        out_type=jax.ShapeDtypeStruct((BATCH * SEQ, EMBED), jnp.float32),
        scratch_types=[
            pltpu.VMEM((_PER_W,), jnp.int32),
            pltpu.VMEM((_PER_W,), jnp.int32),
            pltpu.VMEM((_GROUP, EMBED), jnp.float32),
            pltpu.VMEM((_GROUP, EMBED), jnp.float32),
            pltpu.SemaphoreType.DMA,
            pltpu.SemaphoreType.DMA,
            pltpu.SemaphoreType.DMA,
            pltpu.SemaphoreType.DMA,
            pltpu.SemaphoreType.DMA,
            pltpu.SemaphoreType.DMA,
        ],
        compiler_params=pltpu.CompilerParams(use_tc_tiling_on_sc=False),
    )
    def k(x_hbm, tok_hbm, pos_hbm, pidx_hbm, out_hbm,
          idx_v, pidx_v, buf0, buf1, st0, st1, sp0, sp1, so0, so1):
        wid = lax.axis_index("s") * _NC + lax.axis_index("c")
        base = wid * _PER_W
        pltpu.sync_copy(x_hbm.at[pl.ds(base, _PER_W)], idx_v)
        pltpu.sync_copy(pidx_hbm, pidx_v)

        bufs = (buf0, buf1)
        sems_t = (st0, st1)
        sems_p = (sp0, sp1)
        sems_o = (so0, so1)

        def fire_tok(g, b):
            return [
                pltpu.async_copy(
                    tok_hbm.at[idx_v.at[pl.ds(g * _GROUP + t * _TR, _TR)]],
                    bufs[b].at[pl.ds(t * _TR, _TR)],
                    sems_t[b],
                )
                for t in range(_NTR)
            ]

        def fire_pos(g, b):
            return [
                pltpu.async_copy(
                    pos_hbm.at[pidx_v.at[pl.ds(g * _GROUP + t * _TR, _TR)]],
                    bufs[b].at[pl.ds(t * _TR, _TR)],
                    sems_p[b],
                    add=True,
                )
                for t in range(_NTR)
            ]

        def fire_out(g, b):
            pltpu.async_copy(
                bufs[b], out_hbm.at[pl.ds(base + g * _GROUP, _GROUP)], sems_o[b]
            )

        def drain_out(b):
            pltpu.make_async_copy(
                bufs[b], out_hbm.at[pl.ds(base, _GROUP)], sems_o[b]
            ).wait()

        def round_body(i, carry):
            g0 = i * _NSLOT

            @pl.when(i > 0)
            def _():
                drain_out(0)

            d_t0 = fire_tok(g0, 0)

            @pl.when(i > 0)
            def _():
                drain_out(1)

            d_t1 = fire_tok(g0 + 1, 1)
            for d in d_t0:
                d.wait()
            d_p0 = fire_pos(g0, 0)
            for d in d_t1:
                d.wait()
            d_p1 = fire_pos(g0 + 1, 1)
            for d in d_p0:
                d.wait()
            fire_out(g0, 0)
            for d in d_p1:
                d.wait()
            fire_out(g0 + 1, 1)
            return carry

        lax.fori_loop(0, _NG // _NSLOT, round_body, 0)
        drain_out(0)
        drain_out(1)

    return k(x.reshape(BATCH * SEQ), token_table, pos_table, _POS_IDX)


def _mask_body(x_ref, m_ref):
    m_ref[...] = x_ref[...] != 0


def _mask(x):
    return pl.pallas_call(
        _mask_body,
        out_shape=jax.ShapeDtypeStruct((BATCH, SEQ), jnp.bool_),
        grid=(8,),
        in_specs=[pl.BlockSpec((BATCH // 8, SEQ), lambda i: (i, 0))],
        out_specs=pl.BlockSpec((BATCH // 8, SEQ), lambda i: (i, 0)),
    )(x)


def kernel(x, token_table, pos_table):
    out_flat = _sc_embed(x, token_table, pos_table)
    mask = _mask(x)
    return out_flat.reshape(BATCH, SEQ, EMBED), mask


# speedup vs baseline: 3.8940x; 1.5640x over previous
import functools

import jax
import jax.numpy as jnp
from jax import lax
from jax.experimental import pallas as pl
from jax.experimental.pallas import tpu as pltpu
from jax.experimental.pallas import tpu_sc as plsc

BATCH = 4096
SEQ = 200
EMBED = 64

_NC, _NS = 2, 16
_NW = _NC * _NS
_BROWS_W = BATCH // _NW
_BR = 2
_NG = _BROWS_W // _BR
_NSLOT = 2
_SPLITS = ((0, 104), (104, 96))


def _sc_embed(x, token_table, pos_table):
    mesh = plsc.VectorSubcoreMesh(
        core_axis_name="c", subcore_axis_name="s", num_cores=_NC, num_subcores=_NS
    )

    @functools.partial(
        pl.kernel,
        mesh=mesh,
        out_type=jax.ShapeDtypeStruct((BATCH, SEQ, EMBED), jnp.float32),
        scratch_types=[
            pltpu.VMEM((_BROWS_W, SEQ), jnp.int32),
            pltpu.VMEM_SHARED((SEQ + 8, EMBED), jnp.float32),
            pltpu.VMEM((_BR, SEQ, EMBED), jnp.float32),
            pltpu.VMEM((_BR, SEQ, EMBED), jnp.float32),
            pltpu.SemaphoreType.DMA,
            pltpu.SemaphoreType.DMA,
            pltpu.SemaphoreType.DMA,
            pltpu.SemaphoreType.DMA,
            pltpu.SemaphoreType.DMA,
            pltpu.SemaphoreType.DMA,
        ],
        compiler_params=pltpu.CompilerParams(use_tc_tiling_on_sc=False),
    )
    def k(x_hbm, tok_hbm, pos_hbm, out_hbm,
          idx_v, spos, buf0, buf1, si0, si1, st0, st1, so0, so1):
        sid = lax.axis_index("s")
        wid = sid * _NC + lax.axis_index("c")
        row0 = wid * _BROWS_W
        pltpu.sync_copy(x_hbm.at[pl.ds(row0, _BROWS_W), :], idx_v)

        @pl.when(sid == 0)
        def _():
            pltpu.sync_copy(pos_hbm.at[pl.ds(0, SEQ + 8), :], spos)

        plsc.subcore_barrier()

        bufs = (buf0, buf1)
        sems_i = (si0, si1)
        sems_t = (st0, st1)
        sems_o = (so0, so1)

        def fire_init(b):
            return [
                pltpu.async_copy(
                    spos.at[pl.ds(1, SEQ)], bufs[b].at[br], sems_i[b]
                )
                for br in range(_BR)
            ]

        def fire_tok(g, b):
            return [
                pltpu.async_copy(
                    tok_hbm.at[idx_v.at[g * _BR + br].at[pl.ds(off, ln)]],
                    bufs[b].at[br].at[pl.ds(off, ln)],
                    sems_t[b],
                    add=True,
                )
                for br in range(_BR)
                for off, ln in _SPLITS
            ]

        def fire_out(g, b):
            pltpu.async_copy(
                bufs[b], out_hbm.at[pl.ds(row0 + g * _BR, _BR)], sems_o[b]
            )

        def drain_out(b):
            pltpu.make_async_copy(
                bufs[b], out_hbm.at[pl.ds(row0, _BR)], sems_o[b]
            ).wait()

        def round_body(i, carry):
            g0 = i * _NSLOT

            @pl.when(i > 0)
            def _():
                drain_out(0)

            d_i0 = fire_init(0)

            @pl.when(i > 0)
            def _():
                drain_out(1)

            d_i1 = fire_init(1)
            for d in d_i0:
                d.wait()
            d_t0 = fire_tok(g0, 0)
            for d in d_i1:
                d.wait()
            d_t1 = fire_tok(g0 + 1, 1)
            for d in d_t0:
                d.wait()
            fire_out(g0, 0)
            for d in d_t1:
                d.wait()
            fire_out(g0 + 1, 1)
            return carry

        lax.fori_loop(0, _NG // _NSLOT, round_body, 0)
        drain_out(0)
        drain_out(1)

    return k(x, token_table, pos_table)


def _mask_body(x_ref, m_ref):
    m_ref[...] = x_ref[...] != 0


def _mask(x):
    return pl.pallas_call(
        _mask_body,
        out_shape=jax.ShapeDtypeStruct((BATCH, SEQ), jnp.bool_),
        grid=(8,),
        in_specs=[pl.BlockSpec((BATCH // 8, SEQ), lambda i: (i, 0))],
        out_specs=pl.BlockSpec((BATCH // 8, SEQ), lambda i: (i, 0)),
    )(x)


def kernel(x, token_table, pos_table):
    out = _sc_embed(x, token_table, pos_table)
    mask = _mask(x)
    return out, mask


# speedup vs baseline: 4.1982x; 1.0781x over previous
import functools

import jax
import jax.numpy as jnp
from jax import lax
from jax.experimental import pallas as pl
from jax.experimental.pallas import tpu as pltpu
from jax.experimental.pallas import tpu_sc as plsc

BATCH = 4096
SEQ = 200
EMBED = 64

_NC, _NS = 2, 16
_NW = _NC * _NS
_BROWS_W = BATCH // _NW
_BR = 1
_NG = _BROWS_W // _BR
_NSLOT = 4
_SPLITS = ((0, 104), (104, 96))


def _sc_embed(x, token_table, pos_table):
    mesh = plsc.VectorSubcoreMesh(
        core_axis_name="c", subcore_axis_name="s", num_cores=_NC, num_subcores=_NS
    )

    @functools.partial(
        pl.kernel,
        mesh=mesh,
        out_type=jax.ShapeDtypeStruct((BATCH, SEQ, EMBED), jnp.float32),
        scratch_types=[
            pltpu.VMEM((_BROWS_W, SEQ), jnp.int32),
            pltpu.VMEM_SHARED((SEQ + 8, EMBED), jnp.float32),
        ]
        + [pltpu.VMEM((_BR, SEQ, EMBED), jnp.float32)] * _NSLOT
        + [pltpu.SemaphoreType.DMA] * (3 * _NSLOT),
        compiler_params=pltpu.CompilerParams(use_tc_tiling_on_sc=False),
    )
    def k(x_hbm, tok_hbm, pos_hbm, out_hbm, idx_v, spos, *rest):
        bufs = rest[:_NSLOT]
        sems_i = rest[_NSLOT : 2 * _NSLOT]
        sems_t = rest[2 * _NSLOT : 3 * _NSLOT]
        sems_o = rest[3 * _NSLOT : 4 * _NSLOT]
        sid = lax.axis_index("s")
        wid = sid * _NC + lax.axis_index("c")
        row0 = wid * _BROWS_W
        pltpu.sync_copy(x_hbm.at[pl.ds(row0, _BROWS_W), :], idx_v)

        @pl.when(sid == 0)
        def _():
            pltpu.sync_copy(pos_hbm.at[pl.ds(0, SEQ + 8), :], spos)

        plsc.subcore_barrier()

        def fire_init(b):
            return [
                pltpu.async_copy(
                    spos.at[pl.ds(1, SEQ)], bufs[b].at[br], sems_i[b]
                )
                for br in range(_BR)
            ]

        def fire_tok(g, b):
            return [
                pltpu.async_copy(
                    tok_hbm.at[idx_v.at[g * _BR + br].at[pl.ds(off, ln)]],
                    bufs[b].at[br].at[pl.ds(off, ln)],
                    sems_t[b],
                    add=True,
                )
                for br in range(_BR)
                for off, ln in _SPLITS
            ]

        def fire_out(g, b):
            pltpu.async_copy(
                bufs[b], out_hbm.at[pl.ds(row0 + g * _BR, _BR)], sems_o[b]
            )

        def drain_out(b):
            pltpu.make_async_copy(
                bufs[b], out_hbm.at[pl.ds(row0, _BR)], sems_o[b]
            ).wait()

        def round_body(i, carry):
            g0 = i * _NSLOT
            d_i = [None] * _NSLOT
            d_t = [None] * _NSLOT
            for b in range(_NSLOT):

                @pl.when(i > 0)
                def _(b=b):
                    drain_out(b)

                d_i[b] = fire_init(b)
            for b in range(_NSLOT):
                for d in d_i[b]:
                    d.wait()
                d_t[b] = fire_tok(g0 + b, b)
            for b in range(_NSLOT):
                for d in d_t[b]:
                    d.wait()
                fire_out(g0 + b, b)
            return carry

        lax.fori_loop(0, _NG // _NSLOT, round_body, 0)
        for b in range(_NSLOT):
            drain_out(b)

    return k(x, token_table, pos_table)


def _mask_body(x_ref, m_ref):
    m_ref[...] = x_ref[...] != 0


def _mask(x):
    return pl.pallas_call(
        _mask_body,
        out_shape=jax.ShapeDtypeStruct((BATCH, SEQ), jnp.bool_),
        grid=(8,),
        in_specs=[pl.BlockSpec((BATCH // 8, SEQ), lambda i: (i, 0))],
        out_specs=pl.BlockSpec((BATCH // 8, SEQ), lambda i: (i, 0)),
    )(x)


def kernel(x, token_table, pos_table):
    out = _sc_embed(x, token_table, pos_table)
    mask = _mask(x)
    return out, mask
